# Initial kernel scaffold; baseline (speedup 1.0000x reference)
import jax
import jax.numpy as jnp
from jax.experimental import pallas as pl


def kernel(time_idx, positions):
    raise NotImplementedError("write your pallas kernel here")


# speedup vs baseline: 3.5738x; 3.5738x over previous
import functools

import jax
import jax.numpy as jnp
from jax import lax
from jax.experimental import pallas as pl
from jax.experimental.pallas import tpu as pltpu
from jax.experimental.pallas import tpu_sc as plsc

SEQ_LEN = 2048
HIDDEN = 64
NUM_CORES = 2
NUM_SUBCORES = 16
NUM_WORKERS = NUM_CORES * NUM_SUBCORES
CHUNK = 128


@functools.partial(jax.jit, static_argnums=(2,))
def _sc_gather(idx2d, table, total):
  per_w = total // NUM_WORKERS
  n_chunks = per_w // CHUNK
  mesh = plsc.VectorSubcoreMesh(core_axis_name="c", subcore_axis_name="s")

  @functools.partial(
      pl.kernel,
      out_type=jax.ShapeDtypeStruct((total, HIDDEN), jnp.float32),
      mesh=mesh,
      scratch_types=[
          pltpu.VMEM((n_chunks, CHUNK), jnp.int32),
          pltpu.VMEM((CHUNK, HIDDEN), jnp.float32),
          pltpu.SemaphoreType.DMA,
      ],
      compiler_params=pltpu.CompilerParams(use_tc_tiling_on_sc=False),
  )
  def k(idx_hbm, table_hbm, out_hbm, idx_v, rows_v, gsem):
    wid = lax.axis_index("s") * NUM_CORES + lax.axis_index("c")
    base = wid * per_w
    pltpu.sync_copy(idx_hbm.at[pl.ds(wid * n_chunks, n_chunks)], idx_v)

    def step(j, _):
      pltpu.async_copy(table_hbm.at[idx_v.at[j]], rows_v, gsem).wait()
      pltpu.sync_copy(rows_v, out_hbm.at[pl.ds(base + j * CHUNK, CHUNK)])
      return 0

    lax.fori_loop(0, n_chunks, step, 0)

  return k(idx2d, table)


def kernel(time_idx, positions):
  total = time_idx.shape[0] * time_idx.shape[1]
  idx2d = time_idx.reshape(total // CHUNK, CHUNK)
  out = _sc_gather(idx2d, positions, total)
  return out.reshape(time_idx.shape[0], time_idx.shape[1], HIDDEN)


# speedup vs baseline: 4.0113x; 1.1224x over previous
import functools

import jax
import jax.numpy as jnp
from jax import lax
from jax.experimental import pallas as pl
from jax.experimental.pallas import tpu as pltpu
from jax.experimental.pallas import tpu_sc as plsc

SEQ_LEN = 2048
HIDDEN = 64
NUM_CORES = 2
NUM_SUBCORES = 16
NUM_WORKERS = NUM_CORES * NUM_SUBCORES
CHUNK = 128


@functools.partial(jax.jit, static_argnums=(2,))
def _sc_gather(idx2d, table, total):
  per_w = total // NUM_WORKERS
  n_chunks = per_w // CHUNK
  assert per_w % CHUNK == 0 and n_chunks % 4 == 0 and n_chunks >= 8
  mesh = plsc.VectorSubcoreMesh(core_axis_name="c", subcore_axis_name="s")

  @functools.partial(
      pl.kernel,
      out_type=jax.ShapeDtypeStruct((total, HIDDEN), jnp.float32),
      mesh=mesh,
      scratch_types=[
          pltpu.VMEM((n_chunks, CHUNK), jnp.int32),
          pltpu.VMEM((4, CHUNK, HIDDEN), jnp.float32),
          [pltpu.SemaphoreType.DMA] * 4,
          [pltpu.SemaphoreType.DMA] * 4,
      ],
      compiler_params=pltpu.CompilerParams(use_tc_tiling_on_sc=False),
  )
  def k(idx_hbm, table_hbm, out_hbm, idx_v, rows_v, gsem, osem):
    wid = lax.axis_index("s") * NUM_CORES + lax.axis_index("c")
    base = wid * per_w
    pltpu.sync_copy(idx_hbm.at[pl.ds(wid * n_chunks, n_chunks)], idx_v)

    def gather_start(j, s):
      pltpu.async_copy(table_hbm.at[idx_v.at[j]], rows_v.at[s], gsem[s])

    def gather_wait(j, s):
      pltpu.make_async_copy(
          table_hbm.at[idx_v.at[j]], rows_v.at[s], gsem[s]
      ).wait()

    def store_start(j, s):
      pltpu.async_copy(
          rows_v.at[s], out_hbm.at[pl.ds(base + j * CHUNK, CHUNK)], osem[s]
      )

    def store_wait(j, s):
      pltpu.make_async_copy(
          rows_v.at[s], out_hbm.at[pl.ds(base + j * CHUNK, CHUNK)], osem[s]
      ).wait()

    gather_start(0, 0)
    gather_start(1, 1)
    for s in range(4):
      if s >= 2:
        store_wait(s - 2, s - 2)
        gather_start(s + 2, (s + 2) % 4)
      else:
        gather_start(s + 2, s + 2)
      gather_wait(s, s)
      store_start(s, s)

    def group(p, _):
      j0 = p * 4
      for s in range(4):
        j = j0 + s
        t = (s + 2) % 4
        store_wait(j - 2, t)
        gather_start(j + 2, t)
        gather_wait(j, s)
        store_start(j, s)
      return 0

    lax.fori_loop(1, n_chunks // 4 - 1, group, 0)

    j0 = n_chunks - 4
    for s in range(4):
      j = j0 + s
      t = (s + 2) % 4
      store_wait(j - 2, t)
      if s < 2:
        gather_start(j + 2, t)
      gather_wait(j, s)
      store_start(j, s)
    store_wait(n_chunks - 2, 2)
    store_wait(n_chunks - 1, 3)

  return k(idx2d, table)


def kernel(time_idx, positions):
  total = time_idx.shape[0] * time_idx.shape[1]
  idx2d = time_idx.reshape(total // CHUNK, CHUNK)
  out = _sc_gather(idx2d, positions, total)
  return out.reshape(time_idx.shape[0], time_idx.shape[1], HIDDEN)
